# Initial kernel scaffold; baseline (speedup 1.0000x reference)
import jax
import jax.numpy as jnp
from jax.experimental import pallas as pl


def kernel(x, edge_index, W1, b1, ln1_g, ln1_b, Wqkv, bqkv, Wo, bo, ln2_g, ln2_b, Wfc, bfc, Wp, bp, W2, b2):
    raise NotImplementedError("write your pallas kernel here")


# speedup vs baseline: 13.7429x; 13.7429x over previous
import functools

import jax
import jax.numpy as jnp
import numpy as np
from jax import lax
from jax.experimental import pallas as pl
from jax.experimental.pallas import tpu as pltpu
from jax.experimental.pallas import tpu_sc as plsc

_N = 10000
_E = 320000
_DM = 768
_NP = 10240
_RPAD = 2560
_RPT = _RPAD // 32
_RPT_CS = _RPAD // 16
_ROWS_PT = _NP // 16
_BLK = 512
_G = _NP // _BLK

_mesh = plsc.VectorSubcoreMesh(core_axis_name="c", subcore_axis_name="s")
_sc_params = pltpu.CompilerParams(use_tc_tiling_on_sc=False)


def _sc_agg(table2, src2d, dst2d, dh):
    @functools.partial(
        pl.kernel,
        out_type=jax.ShapeDtypeStruct((2, _NP, dh), jnp.float32),
        mesh=_mesh,
        scratch_types=[
            pltpu.VMEM((_RPT_CS, 128), jnp.int32),
            pltpu.VMEM((_RPT_CS, 128), jnp.int32),
            pltpu.VMEM((128, dh), jnp.float32),
            pltpu.VMEM((128, dh), jnp.float32),
            pltpu.VMEM_SHARED((_NP, dh), jnp.float32),
            pltpu.SemaphoreType.DMA,
        ],
        compiler_params=_sc_params,
    )
    def k(table_hbm, src_hbm, dst_hbm, out_hbm, sidx, didx, rows, zbuf, acc, sem):
        c = lax.axis_index("c")
        s = lax.axis_index("s")

        @pl.loop(0, 128)
        def _(r):
            for kk in range(dh // 16):
                zbuf[r, pl.ds(kk * 16, 16)] = jnp.zeros((16,), jnp.float32)

        @pl.loop(0, 5)
        def _(t):
            pltpu.sync_copy(zbuf, acc.at[pl.ds(s * _ROWS_PT + t * 128, 128)])

        plsc.subcore_barrier()

        pltpu.sync_copy(src_hbm.at[pl.ds(s * _RPT_CS, _RPT_CS)], sidx)
        pltpu.sync_copy(dst_hbm.at[pl.ds(s * _RPT_CS, _RPT_CS)], didx)

        @pl.loop(0, _RPT_CS)
        def _(j):
            pltpu.async_copy(table_hbm.at[c].at[sidx.at[j]], rows, sem).wait()
            pltpu.sync_copy(rows, acc.at[didx.at[j]], add=True)

        plsc.subcore_barrier()

        @pl.loop(0, 5)
        def _(t):
            sl = pl.ds(s * _ROWS_PT + t * 128, 128)
            pltpu.sync_copy(acc.at[sl], rows)
            pltpu.sync_copy(rows, out_hbm.at[c, sl])

    return k(table2, src2d, dst2d)


def _sc_deg(dst2d):
    @functools.partial(
        pl.kernel,
        out_type=jax.ShapeDtypeStruct((2, _NP), jnp.float32),
        mesh=_mesh,
        scratch_types=[
            pltpu.VMEM((_RPT, 128), jnp.int32),
            pltpu.VMEM((128,), jnp.float32),
            pltpu.VMEM((128,), jnp.float32),
            pltpu.VMEM_SHARED((_NP,), jnp.float32),
        ],
        compiler_params=_sc_params,
    )
    def k(dst_hbm, out_hbm, didx, ones, buf, acc):
        c = lax.axis_index("c")
        s = lax.axis_index("s")
        wid = c * 16 + s

        for kk in range(8):
            ones[pl.ds(kk * 16, 16)] = jnp.ones((16,), jnp.float32)
            buf[pl.ds(kk * 16, 16)] = jnp.zeros((16,), jnp.float32)

        @pl.loop(0, 5)
        def _(t):
            pltpu.sync_copy(buf, acc.at[pl.ds(s * _ROWS_PT + t * 128, 128)])

        plsc.subcore_barrier()

        pltpu.sync_copy(dst_hbm.at[pl.ds(wid * _RPT, _RPT)], didx)

        @pl.loop(0, _RPT)
        def _(j):
            pltpu.sync_copy(ones, acc.at[didx.at[j]], add=True)

        plsc.subcore_barrier()

        @pl.loop(0, 5)
        def _(t):
            sl = pl.ds(s * _ROWS_PT + t * 128, 128)
            pltpu.sync_copy(acc.at[sl], buf)
            pltpu.sync_copy(buf, out_hbm.at[c, sl])

    return k(dst2d)


def _ln(h, g, b):
    mu = jnp.mean(h, axis=1, keepdims=True)
    dd = h - mu
    var = jnp.mean(dd * dd, axis=1, keepdims=True)
    return dd * lax.rsqrt(var + 1e-5) * g + b


def _gelu_new(h):
    c = np.sqrt(2.0 / np.pi).astype(np.float32)
    return 0.5 * h * (1.0 + jnp.tanh(c * (h + 0.044715 * h * h * h)))


def _bf(a):
    return a.astype(jnp.bfloat16)


def _dot(a, w):
    return jnp.dot(_bf(a), w, preferred_element_type=jnp.float32)


def _tc_big_body(pr, xr, disr, w1, b1, g1, e1, wv, bv, wo, bo, g2, e2,
                 wfc, bfc, wp, bp, w2, o_ref):
    dis = disr[...]
    u = dis * pr[...] + (dis * dis) * xr[...]
    h = jax.nn.relu(_dot(u, w1[...]) + b1[...])
    for l in range(2):
        a = _ln(h, g1[l], e1[l])
        v = _dot(a, wv[l]) + bv[l]
        h = h + _dot(v, wo[l]) + bo[l]
        m = _ln(h, g2[l], e2[l])
        f = _gelu_new(_dot(m, wfc[l]) + bfc[l])
        h = h + _dot(f, wp[l]) + bp[l]
    r = jax.nn.relu(h)
    o_ref[...] = _dot(r, w2[...]) * dis


def _tc_fin_body(qr, hwdr, disr, b2r, o_ref):
    dis = disr[...]
    z = dis * (qr[...] + hwdr[...]) + b2r[...]
    col = lax.broadcasted_iota(jnp.int32, z.shape, 1)
    zm = jnp.where(col < 40, z, -1e30)
    mx = jnp.max(zm, axis=1, keepdims=True)
    lse = jnp.log(jnp.sum(jnp.exp(zm - mx), axis=1, keepdims=True)) + mx
    o_ref[...] = z - lse


def _row_spec(d):
    return pl.BlockSpec((_BLK, d), lambda i: (i, 0))


def _full(shape):
    nd = len(shape)
    return pl.BlockSpec(shape, lambda i, _nd=nd: (0,) * _nd)


def _part_spec(j, d):
    return pl.BlockSpec((1, _BLK, d), lambda i, _j=j: (_j, i, 0))


def kernel(x, edge_index, W1, b1, ln1_g, ln1_b, Wqkv, bqkv, Wo, bo,
           ln2_g, ln2_b, Wfc, bfc, Wp, bp, W2, b2):
    f32 = jnp.float32
    src, dst = edge_index[0], edge_index[1]
    pad_e = _RPAD * 128 - _E
    padv = jnp.full((pad_e,), _N, jnp.int32)
    src2d = jnp.concatenate([src, padv]).reshape(_RPAD, 128)
    dst2d = jnp.concatenate([dst, padv]).reshape(_RPAD, 128)
    xp = jnp.pad(x, ((0, _NP - _N), (0, 0)))

    degp = _sc_deg(dst2d)
    dis = lax.rsqrt(degp[0] + degp[1] + 1.0)
    dis2 = dis[:, None]
    hs0 = xp * dis2

    hs0s = hs0.reshape(_NP, 2, 64).transpose(1, 0, 2)
    ap = _sc_agg(hs0s, src2d, dst2d, 64)
    agg = ap.transpose(1, 0, 2).reshape(_NP, 128)

    Wv = _bf(Wqkv[:, :, 2 * _DM:])
    bv = bqkv[:, 2 * _DM:]
    W2p = _bf(jnp.pad(W2, ((0, 0), (0, 24))))
    b2p = jnp.pad(b2, (0, 24))[None]

    hwd = pl.pallas_call(
        _tc_big_body,
        grid=(_G,),
        in_specs=[
            _row_spec(128),
            _row_spec(128), _row_spec(1),
            _full((128, _DM)), _full((1, _DM)),
            _full((2, _DM)), _full((2, _DM)),
            _full((2, _DM, _DM)), _full((2, _DM)),
            _full((2, _DM, _DM)), _full((2, _DM)),
            _full((2, _DM)), _full((2, _DM)),
            _full((2, _DM, 3072)), _full((2, 3072)),
            _full((2, 3072, _DM)), _full((2, _DM)),
            _full((_DM, 64)),
        ],
        out_specs=_row_spec(64),
        out_shape=jax.ShapeDtypeStruct((_NP, 64), f32),
    )(agg, xp, dis2, _bf(W1), b1[None], ln1_g, ln1_b, Wv, bv,
      _bf(Wo), bo, ln2_g, ln2_b, _bf(Wfc), bfc, _bf(Wp), bp, W2p)

    hwds = hwd.reshape(_NP, 2, 32).transpose(1, 0, 2)
    qp = _sc_agg(hwds, src2d, dst2d, 32)
    q = qp.transpose(1, 0, 2).reshape(_NP, 64)

    out64 = pl.pallas_call(
        _tc_fin_body,
        grid=(_G,),
        in_specs=[
            _row_spec(64),
            _row_spec(64), _row_spec(1), _full((1, 64)),
        ],
        out_specs=_row_spec(64),
        out_shape=jax.ShapeDtypeStruct((_NP, 64), f32),
    )(q, hwd, dis2, b2p)

    return out64[:_N, :40]


# speedup vs baseline: 16.7004x; 1.2152x over previous
import functools

import jax
import jax.numpy as jnp
import numpy as np
from jax import lax
from jax.experimental import pallas as pl
from jax.experimental.pallas import tpu as pltpu
from jax.experimental.pallas import tpu_sc as plsc

_N = 10000
_E = 320000
_DM = 768
_NP = 10240
_RPAD = 2560
_RPT = _RPAD // 32
_RPT_CS = _RPAD // 16
_ROWS_PT = _NP // 16
_BLK = 512
_G = _NP // _BLK

_mesh = plsc.VectorSubcoreMesh(core_axis_name="c", subcore_axis_name="s")
_sc_params = pltpu.CompilerParams(use_tc_tiling_on_sc=False)


def _sc_agg(table2, src2d, dst2d, dh):
    @functools.partial(
        pl.kernel,
        out_type=jax.ShapeDtypeStruct((2, _NP, dh), jnp.float32),
        mesh=_mesh,
        scratch_types=[
            pltpu.VMEM((_RPT_CS, 128), jnp.int32),
            pltpu.VMEM((_RPT_CS, 128), jnp.int32),
            pltpu.VMEM((4, 128, dh), jnp.float32),
            pltpu.VMEM((128, dh), jnp.float32),
            pltpu.VMEM_SHARED((_NP, dh), jnp.float32),
            pltpu.SemaphoreType.DMA,
            pltpu.SemaphoreType.DMA,
            pltpu.SemaphoreType.DMA,
            pltpu.SemaphoreType.DMA,
        ],
        compiler_params=_sc_params,
    )
    def k(table_hbm, src_hbm, dst_hbm, out_hbm, sidx, didx, rows4, zbuf, acc,
          sem0, sem1, sem2, sem3):
        c = lax.axis_index("c")
        s = lax.axis_index("s")
        sems = [sem0, sem1, sem2, sem3]

        @pl.loop(0, 128)
        def _(r):
            for kk in range(dh // 16):
                zbuf[r, pl.ds(kk * 16, 16)] = jnp.zeros((16,), jnp.float32)

        @pl.loop(0, 5)
        def _(t):
            pltpu.sync_copy(zbuf, acc.at[pl.ds(s * _ROWS_PT + t * 128, 128)])

        plsc.subcore_barrier()

        pltpu.sync_copy(src_hbm.at[pl.ds(s * _RPT_CS, _RPT_CS)], sidx)
        pltpu.sync_copy(dst_hbm.at[pl.ds(s * _RPT_CS, _RPT_CS)], didx)

        for b in range(4):
            pltpu.async_copy(table_hbm.at[c].at[sidx.at[b]], rows4.at[b],
                             sems[b])

        @pl.loop(0, _RPT_CS, step=4)
        def _(j):
            for b in range(4):
                g = j + b
                pltpu.make_async_copy(table_hbm.at[c].at[sidx.at[g]],
                                      rows4.at[b], sems[b]).wait()
                pltpu.sync_copy(rows4.at[b], acc.at[didx.at[g]], add=True)

                @pl.when(g + 4 < _RPT_CS)
                def _():
                    pltpu.async_copy(table_hbm.at[c].at[sidx.at[g + 4]],
                                     rows4.at[b], sems[b])

        plsc.subcore_barrier()

        @pl.loop(0, 5)
        def _(t):
            sl = pl.ds(s * _ROWS_PT + t * 128, 128)
            pltpu.sync_copy(acc.at[sl], zbuf)
            pltpu.sync_copy(zbuf, out_hbm.at[c, sl])

    return k(table2, src2d, dst2d)


def _sc_deg(dst2d):
    @functools.partial(
        pl.kernel,
        out_type=jax.ShapeDtypeStruct((2, _NP), jnp.float32),
        mesh=_mesh,
        scratch_types=[
            pltpu.VMEM((_RPT, 128), jnp.int32),
            pltpu.VMEM((128,), jnp.float32),
            pltpu.VMEM((128,), jnp.float32),
            pltpu.VMEM_SHARED((_NP,), jnp.float32),
        ],
        compiler_params=_sc_params,
    )
    def k(dst_hbm, out_hbm, didx, ones, buf, acc):
        c = lax.axis_index("c")
        s = lax.axis_index("s")
        wid = c * 16 + s

        for kk in range(8):
            ones[pl.ds(kk * 16, 16)] = jnp.ones((16,), jnp.float32)
            buf[pl.ds(kk * 16, 16)] = jnp.zeros((16,), jnp.float32)

        @pl.loop(0, 5)
        def _(t):
            pltpu.sync_copy(buf, acc.at[pl.ds(s * _ROWS_PT + t * 128, 128)])

        plsc.subcore_barrier()

        pltpu.sync_copy(dst_hbm.at[pl.ds(wid * _RPT, _RPT)], didx)

        @pl.loop(0, _RPT)
        def _(j):
            pltpu.sync_copy(ones, acc.at[didx.at[j]], add=True)

        plsc.subcore_barrier()

        @pl.loop(0, 5)
        def _(t):
            sl = pl.ds(s * _ROWS_PT + t * 128, 128)
            pltpu.sync_copy(acc.at[sl], buf)
            pltpu.sync_copy(buf, out_hbm.at[c, sl])

    return k(dst2d)


def _ln(h, g, b):
    mu = jnp.mean(h, axis=1, keepdims=True)
    dd = h - mu
    var = jnp.mean(dd * dd, axis=1, keepdims=True)
    return dd * lax.rsqrt(var + 1e-5) * g + b


def _gelu_new(h):
    c = np.sqrt(2.0 / np.pi).astype(np.float32)
    return 0.5 * h * (1.0 + jnp.tanh(c * (h + 0.044715 * h * h * h)))


def _bf(a):
    return a.astype(jnp.bfloat16)


def _dot(a, w):
    return jnp.dot(_bf(a), w, preferred_element_type=jnp.float32)


def _tc_big_body(pr, xr, disr, w1, b1, g1, e1, wv, bv, wo, bo, g2, e2,
                 wfc, bfc, wp, bp, w2, o_ref):
    dis = disr[...]
    u = dis * pr[...] + (dis * dis) * xr[...]
    h = jax.nn.relu(_dot(u, w1[...]) + b1[...])
    for l in range(2):
        a = _ln(h, g1[l], e1[l])
        v = _dot(a, wv[l]) + bv[l]
        h = h + _dot(v, wo[l]) + bo[l]
        m = _ln(h, g2[l], e2[l])
        f = _gelu_new(_dot(m, wfc[l]) + bfc[l])
        h = h + _dot(f, wp[l]) + bp[l]
    r = jax.nn.relu(h)
    o_ref[...] = _dot(r, w2[...]) * dis


def _tc_fin_body(qr, hwdr, disr, b2r, o_ref):
    dis = disr[...]
    z = dis * (qr[...] + hwdr[...]) + b2r[...]
    col = lax.broadcasted_iota(jnp.int32, z.shape, 1)
    zm = jnp.where(col < 40, z, -1e30)
    mx = jnp.max(zm, axis=1, keepdims=True)
    lse = jnp.log(jnp.sum(jnp.exp(zm - mx), axis=1, keepdims=True)) + mx
    o_ref[...] = z - lse


def _row_spec(d):
    return pl.BlockSpec((_BLK, d), lambda i: (i, 0))


def _full(shape):
    nd = len(shape)
    return pl.BlockSpec(shape, lambda i, _nd=nd: (0,) * _nd)


def _part_spec(j, d):
    return pl.BlockSpec((1, _BLK, d), lambda i, _j=j: (_j, i, 0))


def kernel(x, edge_index, W1, b1, ln1_g, ln1_b, Wqkv, bqkv, Wo, bo,
           ln2_g, ln2_b, Wfc, bfc, Wp, bp, W2, b2):
    f32 = jnp.float32
    src, dst = edge_index[0], edge_index[1]
    pad_e = _RPAD * 128 - _E
    padv = jnp.full((pad_e,), _N, jnp.int32)
    src2d = jnp.concatenate([src, padv]).reshape(_RPAD, 128)
    dst2d = jnp.concatenate([dst, padv]).reshape(_RPAD, 128)
    xp = jnp.pad(x, ((0, _NP - _N), (0, 0)))

    degp = _sc_deg(dst2d)
    dis = lax.rsqrt(degp[0] + degp[1] + 1.0)
    dis2 = dis[:, None]
    hs0 = xp * dis2

    hs0s = hs0.reshape(_NP, 2, 64).transpose(1, 0, 2)
    ap = _sc_agg(hs0s, src2d, dst2d, 64)
    agg = ap.transpose(1, 0, 2).reshape(_NP, 128)

    Wv = _bf(Wqkv[:, :, 2 * _DM:])
    bv = bqkv[:, 2 * _DM:]
    W2p = _bf(jnp.pad(W2, ((0, 0), (0, 24))))
    b2p = jnp.pad(b2, (0, 24))[None]

    hwd = pl.pallas_call(
        _tc_big_body,
        grid=(_G,),
        in_specs=[
            _row_spec(128),
            _row_spec(128), _row_spec(1),
            _full((128, _DM)), _full((1, _DM)),
            _full((2, _DM)), _full((2, _DM)),
            _full((2, _DM, _DM)), _full((2, _DM)),
            _full((2, _DM, _DM)), _full((2, _DM)),
            _full((2, _DM)), _full((2, _DM)),
            _full((2, _DM, 3072)), _full((2, 3072)),
            _full((2, 3072, _DM)), _full((2, _DM)),
            _full((_DM, 64)),
        ],
        out_specs=_row_spec(64),
        out_shape=jax.ShapeDtypeStruct((_NP, 64), f32),
    )(agg, xp, dis2, _bf(W1), b1[None], ln1_g, ln1_b, Wv, bv,
      _bf(Wo), bo, ln2_g, ln2_b, _bf(Wfc), bfc, _bf(Wp), bp, W2p)

    hwds = hwd.reshape(_NP, 2, 32).transpose(1, 0, 2)
    qp = _sc_agg(hwds, src2d, dst2d, 32)
    q = qp.transpose(1, 0, 2).reshape(_NP, 64)

    out64 = pl.pallas_call(
        _tc_fin_body,
        grid=(_G,),
        in_specs=[
            _row_spec(64),
            _row_spec(64), _row_spec(1), _full((1, 64)),
        ],
        out_specs=_row_spec(64),
        out_shape=jax.ShapeDtypeStruct((_NP, 64), f32),
    )(q, hwd, dis2, b2p)

    return out64[:_N, :40]
